# baseline (speedup 1.0000x reference)
import functools

import jax
import jax.numpy as jnp
from jax import lax
from jax.experimental import pallas as pl
from jax.experimental.pallas import tpu as pltpu
from jax.experimental.pallas import tpu_sc as plsc

D_MODEL = 1024
SCALE = 32.0
LANES = 16

NC = 2
NS = 16
NW = NC * NS

B_TOTAL = 4 * 4096
B_PER_W = B_TOTAL // NW
CHUNK = 16
NCH = B_PER_W // CHUNK
NBI = 4
NBO = 2
PER = max(NBI, NBO)
assert NCH % PER == 0 and PER % min(NBI, NBO) == 0 and NCH >= 2 * PER

_mesh = plsc.VectorSubcoreMesh(core_axis_name="c", subcore_axis_name="s")


def _scale_chunk(src, dst):
  @plsc.parallel_loop(0, CHUNK)
  def _row(r):
    @plsc.parallel_loop(0, D_MODEL // LANES, unroll=16)
    def _col(i):
      sl = pl.ds(i * LANES, LANES)
      dst[r, sl] = src[r, sl] * SCALE


@functools.partial(
    pl.kernel,
    out_type=jax.ShapeDtypeStruct((B_TOTAL, D_MODEL), jnp.float32),
    mesh=_mesh,
    scratch_types=(
        [pltpu.VMEM((NCH, CHUNK), jnp.int32)]
        + [pltpu.VMEM((CHUNK, D_MODEL), jnp.float32)] * (NBI + NBO)
        + [pltpu.SemaphoreType.DMA] * (NBI + NBO)
    ),
)
def _emb_lookup(x_hbm, table_hbm, out_hbm, idx_v, *rest):
  ins = rest[:NBI]
  outs = rest[NBI:NBI + NBO]
  gsems = rest[NBI + NBO:2 * NBI + NBO]
  ssems = rest[2 * NBI + NBO:]

  wid = lax.axis_index("s") * NC + lax.axis_index("c")
  base = wid * B_PER_W

  pltpu.sync_copy(x_hbm.at[wid], idx_v)

  for b in range(NBI):
    pltpu.async_copy(table_hbm.at[idx_v.at[b]], ins[b], gsems[b])

  for j in range(PER):
    bi, bo = j % NBI, j % NBO
    pltpu.make_async_copy(
        table_hbm.at[idx_v.at[bi]], ins[bi], gsems[bi]).wait()
    if j >= NBO:
      pltpu.make_async_copy(
          outs[bo], out_hbm.at[pl.ds(base, CHUNK)], ssems[bo]).wait()
    _scale_chunk(ins[bi], outs[bo])
    pltpu.async_copy(
        outs[bo], out_hbm.at[pl.ds(base + j * CHUNK, CHUNK)], ssems[bo])
    pltpu.async_copy(table_hbm.at[idx_v.at[NBI + j]], ins[bi], gsems[bi])

  def group(g, _):
    for b in range(PER):
      j = g * PER + b
      bi, bo = b % NBI, b % NBO
      pltpu.make_async_copy(
          table_hbm.at[idx_v.at[j]], ins[bi], gsems[bi]).wait()
      pltpu.make_async_copy(
          outs[bo], out_hbm.at[pl.ds(base, CHUNK)], ssems[bo]).wait()
      _scale_chunk(ins[bi], outs[bo])
      pltpu.async_copy(
          outs[bo], out_hbm.at[pl.ds(base + j * CHUNK, CHUNK)], ssems[bo])
      nj = j + NBI

      @pl.when(nj < NCH)
      def _():
        pltpu.async_copy(table_hbm.at[idx_v.at[nj]], ins[bi], gsems[bi])
    return 0

  lax.fori_loop(1, NCH // PER, group, 0)

  for b in range(NBO):
    pltpu.make_async_copy(
        outs[b], out_hbm.at[pl.ds(base, CHUNK)], ssems[b]).wait()


def kernel(x, table):
  xf = x.astype(jnp.int32).reshape(NW, NCH, CHUNK)
  out = _emb_lookup(xf, table)
  return out.reshape(x.shape + (D_MODEL,))


# speedup vs baseline: 1.0082x; 1.0082x over previous
import functools

import jax
import jax.numpy as jnp
from jax import lax
from jax.experimental import pallas as pl
from jax.experimental.pallas import tpu as pltpu
from jax.experimental.pallas import tpu_sc as plsc

D_MODEL = 1024
SCALE = 32.0
LANES = 16

NC = 2
NS = 16
NW = NC * NS

B_TOTAL = 4 * 4096
B_PER_W = B_TOTAL // NW
CHUNK = 16
NCH = B_PER_W // CHUNK
NBI = 4
NBO = 2
PER = max(NBI, NBO)
assert NCH % PER == 0 and PER % min(NBI, NBO) == 0 and NCH >= 2 * PER

_mesh = plsc.VectorSubcoreMesh(core_axis_name="c", subcore_axis_name="s")


def _scale_chunk(src, dst):
  @plsc.parallel_loop(0, CHUNK * (D_MODEL // LANES), unroll=8)
  def _elem(t):
    r = t >> 6
    sl = pl.ds((t & 63) * LANES, LANES)
    dst[r, sl] = src[r, sl] * SCALE


@functools.partial(
    pl.kernel,
    out_type=jax.ShapeDtypeStruct((B_TOTAL, D_MODEL), jnp.float32),
    mesh=_mesh,
    scratch_types=(
        [pltpu.VMEM((NCH, CHUNK), jnp.int32)]
        + [pltpu.VMEM((CHUNK, D_MODEL), jnp.float32)] * (NBI + NBO)
        + [pltpu.SemaphoreType.DMA] * (NBI + NBO)
    ),
)
def _emb_lookup(x_hbm, table_hbm, out_hbm, idx_v, *rest):
  ins = rest[:NBI]
  outs = rest[NBI:NBI + NBO]
  gsems = rest[NBI + NBO:2 * NBI + NBO]
  ssems = rest[2 * NBI + NBO:]

  wid = lax.axis_index("s") * NC + lax.axis_index("c")
  base = wid * B_PER_W

  pltpu.sync_copy(x_hbm.at[wid], idx_v)

  for b in range(NBI):
    pltpu.async_copy(table_hbm.at[idx_v.at[b]], ins[b], gsems[b])

  for j in range(PER):
    bi, bo = j % NBI, j % NBO
    pltpu.make_async_copy(
        table_hbm.at[idx_v.at[bi]], ins[bi], gsems[bi]).wait()
    if j >= NBO:
      pltpu.make_async_copy(
          outs[bo], out_hbm.at[pl.ds(base, CHUNK)], ssems[bo]).wait()
    _scale_chunk(ins[bi], outs[bo])
    pltpu.async_copy(
        outs[bo], out_hbm.at[pl.ds(base + j * CHUNK, CHUNK)], ssems[bo])
    pltpu.async_copy(table_hbm.at[idx_v.at[NBI + j]], ins[bi], gsems[bi])

  def group(g, _):
    for b in range(PER):
      j = g * PER + b
      bi, bo = b % NBI, b % NBO
      pltpu.make_async_copy(
          table_hbm.at[idx_v.at[j]], ins[bi], gsems[bi]).wait()
      pltpu.make_async_copy(
          outs[bo], out_hbm.at[pl.ds(base, CHUNK)], ssems[bo]).wait()
      _scale_chunk(ins[bi], outs[bo])
      pltpu.async_copy(
          outs[bo], out_hbm.at[pl.ds(base + j * CHUNK, CHUNK)], ssems[bo])
      nj = j + NBI

      @pl.when(nj < NCH)
      def _():
        pltpu.async_copy(table_hbm.at[idx_v.at[nj]], ins[bi], gsems[bi])
    return 0

  lax.fori_loop(1, NCH // PER, group, 0)

  for b in range(NBO):
    pltpu.make_async_copy(
        outs[b], out_hbm.at[pl.ds(base, CHUNK)], ssems[b]).wait()


def kernel(x, table):
  xf = x.astype(jnp.int32).reshape(NW, NCH, CHUNK)
  out = _emb_lookup(xf, table)
  return out.reshape(x.shape + (D_MODEL,))


# speedup vs baseline: 1.0097x; 1.0015x over previous
import functools

import jax
import jax.numpy as jnp
from jax import lax
from jax.experimental import pallas as pl
from jax.experimental.pallas import tpu as pltpu
from jax.experimental.pallas import tpu_sc as plsc

D_MODEL = 1024
SCALE = 32.0
LANES = 16

NC = 2
NS = 16
NW = NC * NS

B_TOTAL = 4 * 4096
B_PER_W = B_TOTAL // NW
CHUNK = 16
NCH = B_PER_W // CHUNK
NBI = 4
NBO = 2
PER = max(NBI, NBO)
assert NCH % PER == 0 and PER % min(NBI, NBO) == 0 and NCH >= 2 * PER

_mesh = plsc.VectorSubcoreMesh(core_axis_name="c", subcore_axis_name="s")


def _scale_chunk(src, dst):
  @plsc.parallel_loop(0, CHUNK * (D_MODEL // LANES), unroll=8)
  def _elem(t):
    r = t >> 6
    sl = pl.ds((t & 63) * LANES, LANES)
    dst[r, sl] = src[r, sl] * SCALE


@functools.partial(
    pl.kernel,
    out_type=jax.ShapeDtypeStruct((B_TOTAL, D_MODEL), jnp.float32),
    mesh=_mesh,
    scratch_types=(
        [pltpu.VMEM((NCH, CHUNK), jnp.int32)]
        + [pltpu.VMEM((CHUNK, D_MODEL), jnp.float32)] * (NBI + NBO)
        + [pltpu.SemaphoreType.DMA] * (NBI + NBO)
    ),
)
def _emb_lookup(x_hbm, table_hbm, out_hbm, idx_v, *rest):
  ins = rest[:NBI]
  outs = rest[NBI:NBI + NBO]
  gsems = rest[NBI + NBO:2 * NBI + NBO]
  ssems = rest[2 * NBI + NBO:]

  wid = lax.axis_index("s") * NC + lax.axis_index("c")
  base = wid * B_PER_W

  pltpu.sync_copy(x_hbm.at[wid], idx_v)

  for b in range(NBI):
    pltpu.async_copy(table_hbm.at[idx_v.at[b]], ins[b], gsems[b])

  for j in range(PER):
    bi, bo = j % NBI, j % NBO
    pltpu.make_async_copy(
        table_hbm.at[idx_v.at[bi]], ins[bi], gsems[bi]).wait()
    if j >= NBO:
      pltpu.make_async_copy(
          outs[bo], out_hbm.at[pl.ds(base, CHUNK)], ssems[bo]).wait()
    _scale_chunk(ins[bi], outs[bo])
    pltpu.async_copy(
        outs[bo], out_hbm.at[pl.ds(base + j * CHUNK, CHUNK)], ssems[bo])
    pltpu.async_copy(table_hbm.at[idx_v.at[NBI + j]], ins[bi], gsems[bi])

  def group(g, _):
    for b in range(PER):
      j = g * PER + b
      bi, bo = b % NBI, b % NBO
      pltpu.make_async_copy(
          table_hbm.at[idx_v.at[j]], ins[bi], gsems[bi]).wait()
      pltpu.make_async_copy(
          outs[bo], out_hbm.at[pl.ds(base, CHUNK)], ssems[bo]).wait()
      _scale_chunk(ins[bi], outs[bo])
      nj = j + NBI

      @pl.when(nj < NCH)
      def _():
        pltpu.async_copy(table_hbm.at[idx_v.at[nj]], ins[bi], gsems[bi])
      pltpu.async_copy(
          outs[bo], out_hbm.at[pl.ds(base + j * CHUNK, CHUNK)], ssems[bo])
    return 0

  lax.fori_loop(1, NCH // PER, group, 0)

  for b in range(NBO):
    pltpu.make_async_copy(
        outs[b], out_hbm.at[pl.ds(base, CHUNK)], ssems[b]).wait()


def kernel(x, table):
  xf = x.astype(jnp.int32).reshape(NW, NCH, CHUNK)
  out = _emb_lookup(xf, table)
  return out.reshape(x.shape + (D_MODEL,))
